# Initial kernel scaffold; baseline (speedup 1.0000x reference)
import jax
import jax.numpy as jnp
from jax.experimental import pallas as pl


def kernel(x, edge_index, embed_W, embed_b, gcn_W, gat_W, att_src, att_dst, gat_b, fus_W, fus_b, ln_g, ln_b, sae_W, sae_b, r1_W, r1_b, r2_W, r2_b):
    raise NotImplementedError("write your pallas kernel here")


# speedup vs baseline: 22.5927x; 22.5927x over previous
import functools

import jax
import jax.numpy as jnp
from jax import lax
from jax.experimental import pallas as pl
from jax.experimental.pallas import tpu as pltpu
from jax.experimental.pallas import tpu_sc as plsc

N = 10000
E = 320000
EP = 327680
H = 128
HH = 64
HEADS = 4
OUT = 32
DS = 512

NS = 16
NP = 10240
NPT = NP // NS
KC = 128
CC = EP // NS // KC
CB = EP // (2 * NS) // KC
RING = 2

_sc_mesh = plsc.VectorSubcoreMesh(core_axis_name="c", subcore_axis_name="s")
_sc_params = pltpu.CompilerParams(needs_layout_passes=False,
                                  use_tc_tiling_on_sc=False)
f32 = jnp.float32


@functools.partial(
    pl.kernel,
    out_type=jax.ShapeDtypeStruct((2, NP, H), f32),
    mesh=_sc_mesh,
    compiler_params=_sc_params,
    scratch_types=[
        pltpu.VMEM((KC,), jnp.int32),
        pltpu.VMEM((KC,), jnp.int32),
        pltpu.VMEM((KC, 8), f32),
        pltpu.VMEM_SHARED((NP, 8), f32),
        pltpu.SemaphoreType.DMA,
        pltpu.SemaphoreType.DMA,
    ],
)
def _deg_kernel(col_hbm, z128_hbm, deg_hbm, i0, i1, ones_v, deg_sh, is_, ss):
    c = lax.axis_index("c")
    s = lax.axis_index("s")
    w = c * NS + s
    rs = pl.ds(s * NPT, NPT)
    iota = lax.iota(jnp.int32, 16)
    ones16 = jnp.full((16,), 1.0, f32)
    zcol = jnp.zeros((16,), jnp.int32)
    pltpu.sync_copy(z128_hbm.at[pl.ds(0, KC), pl.ds(0, 8)], ones_v)
    for g in range(KC // 16):
        plsc.store_scatter(ones_v, [iota + g * 16, zcol], ones16)
    pltpu.sync_copy(z128_hbm.at[:, pl.ds(0, 8)], deg_sh.at[rs])
    plsc.subcore_barrier()

    ib = [i0, i1]

    def body(j, carry):
        for b in range(2):
            ch = j * 2 + b
            pltpu.sync_copy(col_hbm.at[w, ch], ib[b])
            pltpu.async_copy(ones_v, deg_sh.at[ib[b]], ss, add=True)
        for b in range(2):
            pltpu.make_async_copy(ones_v, deg_sh.at[pl.ds(0, KC)], ss).wait()
        return carry

    lax.fori_loop(0, CB // 2, body, 0)
    plsc.subcore_barrier()
    pltpu.sync_copy(deg_sh.at[rs], deg_hbm.at[c, rs, pl.ds(0, 8)])


BR = 400
GR = N // BR


def _tc_pre_body(x_ref, deg2_ref, ewt_ref, eb_ref, gwt_ref, acat_ref,
                 z_ref, h_ref, xg_ref, h2_ref, asd_ref, disv_ref):
    xb = x_ref[...]
    z = jnp.dot(xb, ewt_ref[...], preferred_element_type=f32) + eb_ref[...]
    hb = jnp.tanh(z)
    xg = jnp.dot(hb, gwt_ref[...], preferred_element_type=f32)
    asdb = jnp.dot(xg, acat_ref[...], preferred_element_type=f32)
    degb = deg2_ref[0, :, 0:8] + deg2_ref[1, :, 0:8]
    dis8 = jnp.where(degb > 0, lax.rsqrt(degb), 0.0)
    z_ref[...] = z
    h_ref[...] = hb
    xg_ref[...] = xg
    h2_ref[...] = dis8[:, 0:1] * hb
    asd_ref[...] = asdb
    disv_ref[...] = dis8


def _tc_pre(x, deg2, ewt, eb, gwt, acat):
    return pl.pallas_call(
        _tc_pre_body,
        grid=(GR,),
        in_specs=[
            pl.BlockSpec((BR, H), lambda i: (i, 0)),
            pl.BlockSpec((2, BR, H), lambda i: (0, i, 0)),
            pl.BlockSpec((H, H), lambda i: (0, 0)),
            pl.BlockSpec((1, H), lambda i: (0, 0)),
            pl.BlockSpec((H, H), lambda i: (0, 0)),
            pl.BlockSpec((H, 8), lambda i: (0, 0)),
        ],
        out_specs=[
            pl.BlockSpec((BR, H), lambda i: (i, 0)),
            pl.BlockSpec((BR, H), lambda i: (i, 0)),
            pl.BlockSpec((BR, H), lambda i: (i, 0)),
            pl.BlockSpec((BR, H), lambda i: (i, 0)),
            pl.BlockSpec((BR, 8), lambda i: (i, 0)),
            pl.BlockSpec((BR, 8), lambda i: (i, 0)),
        ],
        out_shape=[
            jax.ShapeDtypeStruct((N, H), f32),
            jax.ShapeDtypeStruct((N, H), f32),
            jax.ShapeDtypeStruct((N, H), f32),
            jax.ShapeDtypeStruct((N, H), f32),
            jax.ShapeDtypeStruct((N, 8), f32),
            jax.ShapeDtypeStruct((N, 8), f32),
        ],
    )(x, deg2, ewt, eb, gwt, acat)


@functools.partial(
    pl.kernel,
    out_type=[
        jax.ShapeDtypeStruct((EP * 4,), f32),
        jax.ShapeDtypeStruct((2, NP, H), f32),
    ],
    mesh=_sc_mesh,
    compiler_params=_sc_params,
    scratch_types=[
        pltpu.VMEM((NP * 8,), f32),
        pltpu.VMEM((KC,), jnp.int32),
        pltpu.VMEM((KC,), jnp.int32),
        pltpu.VMEM((KC, 8), f32),
        pltpu.VMEM((KC * 4,), f32),
        pltpu.VMEM_SHARED((NP, 8), f32),
        pltpu.SemaphoreType.DMA,
    ],
)
def _ee_kernel(rowr, colr, asdf, z128, ee_hbm, den_hbm,
               asd_v, ir, ic, eeb, eef, den_sh, ss):
    c = lax.axis_index("c")
    s = lax.axis_index("s")
    w = c * NS + s
    rs = pl.ds(s * NPT, NPT)
    pltpu.sync_copy(asdf, asd_v)
    pltpu.sync_copy(z128.at[pl.ds(0, KC), pl.ds(0, 8)], eeb)
    pltpu.sync_copy(z128.at[:, pl.ds(0, 8)], den_sh.at[rs])
    plsc.subcore_barrier()
    iota = lax.iota(jnp.int32, 16)

    def body(j, carry):
        pltpu.sync_copy(rowr.at[w, j], ir)
        pltpu.sync_copy(colr.at[w, j], ic)
        for g in range(KC // 16):
            gsl = pl.ds(g * 16, 16)
            rv = ir[gsl] * 8
            cv = ic[gsl] * 8 + 4
            ide = iota + (g * 16)
            for hh in range(HEADS):
                av = (plsc.load_gather(asd_v, [rv + hh])
                      + plsc.load_gather(asd_v, [cv + hh]))
                ev = jnp.exp(jnp.maximum(av, 0.2 * av))
                plsc.store_scatter(eeb, [ide, jnp.full((16,), hh, jnp.int32)],
                                   ev)
                plsc.store_scatter(eef, [ide * 4 + hh], ev)
        pltpu.sync_copy(eeb, den_sh.at[ic], add=True)
        pltpu.sync_copy(eef, ee_hbm.at[pl.ds((w * CB + j) * KC * 4, KC * 4)])
        return carry

    lax.fori_loop(0, CB, body, 0)
    plsc.subcore_barrier()
    pltpu.sync_copy(den_sh.at[rs], den_hbm.at[c, rs, pl.ds(0, 8)])


@functools.partial(
    pl.kernel,
    out_type=[
        jax.ShapeDtypeStruct((NP, H), f32),
        jax.ShapeDtypeStruct((NP, H), f32),
    ],
    mesh=_sc_mesh,
    compiler_params=_sc_params,
    scratch_types=(
        [pltpu.VMEM((KC,), jnp.int32) for _ in range(RING)]
        + [pltpu.VMEM((KC,), jnp.int32) for _ in range(RING)]
        + [pltpu.VMEM((KC, H), f32) for _ in range(RING)]
        + [pltpu.VMEM((KC, HH), f32) for _ in range(RING)]
        + [pltpu.VMEM((KC * 4,), f32) for _ in range(RING)]
        + [pltpu.VMEM_SHARED((NP, HH), f32)]
        + [pltpu.SemaphoreType.DMA for _ in range(3 * RING)]
    ),
)
def _edge_kernel(rowr, colr, h2p, xgp, eeflat, z128, agg_out, gat_out, *refs):
    gi = list(refs[0:RING])
    si = list(refs[RING:2 * RING])
    gb = list(refs[2 * RING:3 * RING])
    sb = list(refs[3 * RING:4 * RING])
    eb = list(refs[4 * RING:5 * RING])
    acc_sh = refs[5 * RING]
    isem = list(refs[5 * RING + 1:5 * RING + 1 + RING])
    gsem = list(refs[5 * RING + 1 + RING:5 * RING + 1 + 2 * RING])
    ssem = list(refs[5 * RING + 1 + 2 * RING:5 * RING + 1 + 3 * RING])
    c = lax.axis_index("c")
    s = lax.axis_index("s")
    rs = pl.ds(s * NPT, NPT)
    iota = lax.iota(jnp.int32, 16)

    pltpu.sync_copy(z128.at[:, pl.ds(0, HH)], acc_sh.at[rs])
    plsc.subcore_barrier()

    def idx_start(b, ch, gat):
        if gat:
            pltpu.async_copy(rowr.at[s, ch], gi[b], isem[b])
            pltpu.async_copy(colr.at[s, ch], si[b], isem[b])
        else:
            pltpu.async_copy(colr.at[s, ch], gi[b], isem[b])
            pltpu.async_copy(rowr.at[s, ch], si[b], isem[b])

    def idx_wait(b):
        pltpu.make_async_copy(rowr.at[0, 0], gi[b], isem[b]).wait()
        pltpu.make_async_copy(rowr.at[0, 0], si[b], isem[b]).wait()

    def run_phase(src, out_col, gat, heads):
        for b in range(RING):
            idx_start(b, b, gat)

        def body(j, carry):
            for b in range(RING):
                ch = j * RING + b
                idx_wait(b)
                pltpu.async_copy(src.at[gi[b]], gb[b], gsem[b])
                if gat:
                    off = (s * CC + ch) * KC * 4
                    pltpu.async_copy(eeflat.at[pl.ds(off, KC * 4)], eb[b],
                                     gsem[b])
            for b in range(RING):
                ch = j * RING + b
                pltpu.make_async_copy(src.at[pl.ds(0, KC)], gb[b],
                                      gsem[b]).wait()
                if gat:
                    pltpu.make_async_copy(eeflat.at[pl.ds(0, KC * 4)], eb[b],
                                          gsem[b]).wait()

                    def sc_body(q, cy):
                        ev16 = eb[b][pl.ds(q * 16, 16)]
                        for jj in range(4):
                            e = q * 4 + jj
                            for hh in heads:
                                sv = ev16[jj * 4 + hh]
                                for v in range(2):
                                    gco = hh * OUT + v * 16
                                    sco = (hh % 2) * OUT + v * 16
                                    sb[b][e, pl.ds(sco, 16)] = (
                                        gb[b][e, pl.ds(gco, 16)] * sv)
                        return cy

                    lax.fori_loop(0, KC // 4, sc_body, 0)
                else:

                    def cp_body(e, cy):
                        for v in range(4):
                            sb[b][e, pl.ds(v * 16, 16)] = (
                                gb[b][e, pl.ds(out_col + v * 16, 16)])
                        return cy

                    lax.fori_loop(0, KC, cp_body, 0)
                pltpu.async_copy(sb[b], acc_sh.at[si[b]], ssem[b], add=True)
            for b in range(RING):
                ch = j * RING + b
                pltpu.make_async_copy(sb[b], acc_sh.at[pl.ds(0, KC)],
                                      ssem[b]).wait()

                @pl.when(ch + RING < CC)
                def _():
                    idx_start(b, ch + RING, gat)
            return carry

        lax.fori_loop(0, CC // RING, body, 0)
        plsc.subcore_barrier()

    @pl.when(c == 0)
    def _gcn():
        run_phase(h2p, 0, False, (0, 1))
        pltpu.sync_copy(acc_sh.at[rs], agg_out.at[rs, pl.ds(0, HH)])
        pltpu.sync_copy(z128.at[:, pl.ds(0, HH)], acc_sh.at[rs])
        plsc.subcore_barrier()
        run_phase(h2p, HH, False, (0, 1))
        pltpu.sync_copy(acc_sh.at[rs], agg_out.at[rs, pl.ds(HH, HH)])

    @pl.when(c == 1)
    def _gat():
        run_phase(xgp, 0, True, (0, 1))
        pltpu.sync_copy(acc_sh.at[rs], gat_out.at[rs, pl.ds(0, HH)])
        pltpu.sync_copy(z128.at[:, pl.ds(0, HH)], acc_sh.at[rs])
        plsc.subcore_barrier()
        run_phase(xgp, 0, True, (2, 3))
        pltpu.sync_copy(acc_sh.at[rs], gat_out.at[rs, pl.ds(HH, HH)])


def _tc_post_body(h_ref, xg_ref, asd_ref, agg_ref, gat_ref, den_ref, disv_ref,
                  gwt_ref, fwa_ref, fwb_ref, fb_ref, gb_ref,
                  lng_ref, lnb_ref, rm_ref, saew_ref, saeb_ref,
                  r1wt_ref, r1b_ref, r2wt_ref, r2b_ref,
                  lg_ref, hr_ref, zs_ref, zf_ref, xo_ref):
    hb = h_ref[...]
    xg = xg_ref[...]
    asdb = asd_ref[...]
    rm = rm_ref[...]
    dis = disv_ref[...][:, 0:1]
    x_gcn = jnp.dot(dis * agg_ref[...], gwt_ref[...],
                    preferred_element_type=f32)
    e4s = asdb[:, 0:4] + asdb[:, 4:8]
    ee_self = jnp.exp(jnp.maximum(e4s, 0.2 * e4s))
    den_f = (den_ref[0, :, 0:4] + den_ref[1, :, 0:4]) + ee_self + 1e-16
    x_gat = ((gat_ref[...] + jnp.dot(ee_self, rm, preferred_element_type=f32)
              * xg)
             / jnp.dot(den_f, rm, preferred_element_type=f32)) + gb_ref[...]
    x_fused = (jnp.dot(x_gcn, fwa_ref[...], preferred_element_type=f32)
               + jnp.dot(x_gat, fwb_ref[...], preferred_element_type=f32)
               + fb_ref[...])
    pre = x_fused + hb
    mu = jnp.mean(pre, axis=-1, keepdims=True)
    var = jnp.mean((pre - mu) ** 2, axis=-1, keepdims=True)
    ln = (pre - mu) / jnp.sqrt(var + 1e-5) * lng_ref[...] + lnb_ref[...]
    x_out = jnp.maximum(ln, 0.0)
    z_sae = jnp.maximum(jnp.dot(x_out, saew_ref[...],
                                preferred_element_type=f32) + saeb_ref[...],
                        0.0)
    h_recon = lax.dot_general(z_sae, saew_ref[...], (((1,), (1,)), ((), ())),
                              preferred_element_type=f32)
    hid = jnp.maximum(jnp.dot(x_out, r1wt_ref[...],
                              preferred_element_type=f32) + r1b_ref[...], 0.0)
    logits = jax.nn.sigmoid(jnp.dot(hid, r2wt_ref[...],
                                    preferred_element_type=f32) + r2b_ref[...])
    lg_ref[...] = logits
    hr_ref[...] = h_recon
    zs_ref[...] = z_sae
    zf_ref[...] = x_fused
    xo_ref[...] = x_out


def _tc_post(h, xg, asd, agg, gat, den, disv, gwt, fwa, fwb, fb, gbias,
             lng, lnb, rm, saew, saeb, r1wt, r1b, r2wt, r2b):
    full = lambda shp: pl.BlockSpec(shp, lambda i: tuple(0 for _ in shp))
    rowblk = lambda m: pl.BlockSpec((BR, m), lambda i: (i, 0))
    return pl.pallas_call(
        _tc_post_body,
        grid=(GR,),
        in_specs=[
            rowblk(H), rowblk(H), rowblk(8), rowblk(H), rowblk(H),
            pl.BlockSpec((2, BR, H), lambda i: (0, i, 0)), rowblk(8),
            full((H, H)), full((H, H)), full((H, H)),
            full((1, H)), full((1, H)), full((1, H)), full((1, H)),
            full((HEADS, H)), full((H, DS)), full((1, DS)),
            full((H, 64)), full((1, 64)), full((64, 8)), full((1, 8)),
        ],
        out_specs=[
            pl.BlockSpec((BR, 8), lambda i: (i, 0)),
            rowblk(H), rowblk(DS), rowblk(H), rowblk(H),
        ],
        out_shape=[
            jax.ShapeDtypeStruct((N, 8), f32),
            jax.ShapeDtypeStruct((N, H), f32),
            jax.ShapeDtypeStruct((N, DS), f32),
            jax.ShapeDtypeStruct((N, H), f32),
            jax.ShapeDtypeStruct((N, H), f32),
        ],
    )(h, xg, asd, agg, gat, den, disv, gwt, fwa, fwb, fb, gbias,
      lng, lnb, rm, saew, saeb, r1wt, r1b, r2wt, r2b)


def kernel(x, edge_index, embed_W, embed_b, gcn_W, gat_W, att_src, att_dst,
           gat_b, fus_W, fus_b, ln_g, ln_b, sae_W, sae_b, r1_W, r1_b, r2_W,
           r2_b):
    row = edge_index[0]
    col = edge_index[1]

    nd = EP - E
    dummy = N + (jnp.arange(nd, dtype=jnp.int32) % 240)
    rowp = jnp.concatenate([row, dummy])
    colp = jnp.concatenate([col, dummy])
    rowr = rowp.reshape(NS, CC, KC)
    colr = colp.reshape(NS, CC, KC)
    rowr2 = rowp.reshape(2 * NS, CB, KC)
    colr2 = colp.reshape(2 * NS, CB, KC)

    z128 = jnp.zeros((NPT, H), f32)

    deg2 = _deg_kernel(colr2, z128)

    eye4 = jnp.eye(HEADS, dtype=f32)
    acat = jnp.concatenate(
        [(att_src[:, :, None] * eye4[:, None, :]).reshape(H, HEADS),
         (att_dst[:, :, None] * eye4[:, None, :]).reshape(H, HEADS)], axis=1)

    z_embed, h, xg, h2, asd, disv = _tc_pre(
        x, deg2, embed_W.T, embed_b.reshape(1, H), gat_W.T, acat)

    asdf = jnp.zeros((NP * 8,), f32).at[:N * 8].set(asd.reshape(N * 8))
    h2p = jnp.zeros((NP, H), f32).at[:N].set(h2)
    xgp = jnp.zeros((NP, H), f32).at[:N].set(xg)

    eeflat, den = _ee_kernel(rowr2, colr2, asdf, z128)
    agg, gat = _edge_kernel(rowr, colr, h2p, xgp, eeflat, z128)

    rm = jnp.repeat(eye4, OUT, axis=1)
    r2wt = jnp.zeros((64, 8), f32).at[:, 0].set(r2_W[0])
    r2b8 = jnp.zeros((1, 8), f32).at[0, 0].set(r2_b[0])

    logits8, h_recon, z_sae, z_fused, x_out = _tc_post(
        h, xg, asd, agg[:N], gat[:N], den[:, :N], disv,
        gcn_W.T, fus_W[:, :H].T, fus_W[:, H:].T, fus_b.reshape(1, H),
        gat_b.reshape(1, H), ln_g.reshape(1, H), ln_b.reshape(1, H), rm,
        sae_W, sae_b.reshape(1, DS), r1_W.T, r1_b.reshape(1, 64), r2wt, r2b8)

    return (logits8[:, 0:1], h_recon, z_sae, z_embed, z_fused, x_out)
